# Initial kernel scaffold; baseline (speedup 1.0000x reference)
import jax
import jax.numpy as jnp
from jax.experimental import pallas as pl


def kernel(x, edge_index, W1_l, b1_l, W1_r, W2_l, b2_l, W2_r, Wc, bc):
    raise NotImplementedError("write your pallas kernel here")


# speedup vs baseline: 6.3466x; 6.3466x over previous
import functools

import jax
import jax.numpy as jnp
from jax import lax
from jax.experimental import pallas as pl
from jax.experimental.pallas import tpu as pltpu
from jax.experimental.pallas import tpu_sc as plsc

N = 10000
E = 320000
NC = 2
NS = 16
NW = NC * NS
E_PER_W = E // NW
CHUNK = 80
N_CHUNKS = E_PER_W // CHUNK
N_PAD = 10240
ROWS_PER_TILE = N_PAD // NS
ZROWS = 128
CW = 16


def _seg_sum_sc(p, src, dst, d, with_counts):
  mesh = plsc.VectorSubcoreMesh(
      core_axis_name="c", subcore_axis_name="s", num_cores=NC,
      num_subcores=NS)

  out_type = [jax.ShapeDtypeStruct((NC, N_PAD, d), jnp.float32)]
  if with_counts:
    out_type.append(jax.ShapeDtypeStruct((NC, N_PAD, CW), jnp.float32))

  scratch = [
      pltpu.VMEM((CHUNK,), jnp.int32),
      pltpu.VMEM((CHUNK,), jnp.int32),
      pltpu.VMEM((CHUNK, d), jnp.float32),
      pltpu.VMEM((ZROWS, d), jnp.float32),
      pltpu.VMEM_SHARED((N_PAD, d), jnp.float32),
      pltpu.SemaphoreType.DMA,
  ]
  if with_counts:
    scratch += [
        pltpu.VMEM((CHUNK, CW), jnp.float32),
        pltpu.VMEM((ZROWS, CW), jnp.float32),
        pltpu.VMEM_SHARED((N_PAD, CW), jnp.float32),
    ]

  def body(p_hbm, src_hbm, dst_hbm, *rest):
    if with_counts:
      (sums_hbm, cnts_hbm, src_v, dst_v, rows_v, zer_v, acc_sh, sem,
       ones_v, zerc_v, cnt_sh) = rest
    else:
      sums_hbm, src_v, dst_v, rows_v, zer_v, acc_sh, sem = rest
    sid = lax.axis_index("s")
    cid = lax.axis_index("c")
    wid = sid * NC + cid

    def init_row(i, _):
      for j in range(d // 16):
        zer_v[i, pl.ds(j * 16, 16)] = jnp.zeros((16,), jnp.float32)
      if with_counts:
        zerc_v[i, pl.ds(0, 16)] = jnp.zeros((16,), jnp.float32)
      return _
    lax.fori_loop(0, ZROWS, init_row, 0)
    if with_counts:
      def init_ones(i, _):
        ones_v[i, pl.ds(0, 16)] = jnp.ones((16,), jnp.float32)
        return _
      lax.fori_loop(0, CHUNK, init_ones, 0)

    base = sid * ROWS_PER_TILE
    for k in range(ROWS_PER_TILE // ZROWS):
      pltpu.sync_copy(zer_v, acc_sh.at[pl.ds(base + k * ZROWS, ZROWS)])
      if with_counts:
        pltpu.sync_copy(zerc_v, cnt_sh.at[pl.ds(base + k * ZROWS, ZROWS)])
    plsc.subcore_barrier()

    def step(i, _):
      off = wid * E_PER_W + i * CHUNK
      pltpu.sync_copy(src_hbm.at[pl.ds(off, CHUNK)], src_v)
      pltpu.sync_copy(dst_hbm.at[pl.ds(off, CHUNK)], dst_v)
      pltpu.async_copy(p_hbm.at[src_v], rows_v, sem).wait()
      pltpu.sync_copy(rows_v, acc_sh.at[dst_v], add=True)
      if with_counts:
        pltpu.sync_copy(ones_v, cnt_sh.at[dst_v], add=True)
      return _
    lax.fori_loop(0, N_CHUNKS, step, 0)

    plsc.subcore_barrier()
    pltpu.sync_copy(acc_sh.at[pl.ds(base, ROWS_PER_TILE)],
                    sums_hbm.at[cid, pl.ds(base, ROWS_PER_TILE)])
    if with_counts:
      pltpu.sync_copy(cnt_sh.at[pl.ds(base, ROWS_PER_TILE)],
                      cnts_hbm.at[cid, pl.ds(base, ROWS_PER_TILE)])

  fn = pl.kernel(
      body, out_type=out_type, mesh=mesh, scratch_types=scratch,
      compiler_params=pltpu.CompilerParams(use_tc_tiling_on_sc=False))
  return fn(p, src, dst)


_BN = 2000


def _tc_proj(x, wl_t, wr_t):
  d_in, d_out = wl_t.shape

  def body(x_ref, wl_ref, wr_ref, p_ref, r_ref):
    xb = x_ref[...]
    p_ref[...] = jnp.dot(xb, wl_ref[...], preferred_element_type=jnp.float32)
    r_ref[...] = jnp.dot(xb, wr_ref[...], preferred_element_type=jnp.float32)

  return pl.pallas_call(
      body,
      grid=(N // _BN,),
      in_specs=[
          pl.BlockSpec((_BN, d_in), lambda i: (i, 0)),
          pl.BlockSpec((d_in, d_out), lambda i: (0, 0)),
          pl.BlockSpec((d_in, d_out), lambda i: (0, 0)),
      ],
      out_specs=[
          pl.BlockSpec((_BN, d_out), lambda i: (i, 0)),
          pl.BlockSpec((_BN, d_out), lambda i: (i, 0)),
      ],
      out_shape=[
          jax.ShapeDtypeStruct((N, d_out), jnp.float32),
          jax.ShapeDtypeStruct((N, d_out), jnp.float32),
      ],
  )(x, wl_t, wr_t)


def _tc_mid(s1, cnt, r1, b1, w2l_t, w2r_t):
  d1 = r1.shape[1]
  d2 = w2l_t.shape[1]

  def body(s_ref, c_ref, r_ref, b_ref, wl_ref, wr_ref, p_ref, rr_ref):
    sv = s_ref[...]
    cv = c_ref[...]
    s = sv[0] + sv[1]
    cnt_col = cv[0, :, 0:1] + cv[1, :, 0:1]
    mean = s / jnp.maximum(cnt_col, 1.0)
    h1 = jnp.maximum(mean + b_ref[...] + r_ref[...], 0.0)
    p_ref[...] = jnp.dot(h1, wl_ref[...], preferred_element_type=jnp.float32)
    rr_ref[...] = jnp.dot(h1, wr_ref[...], preferred_element_type=jnp.float32)

  return pl.pallas_call(
      body,
      grid=(N // _BN,),
      in_specs=[
          pl.BlockSpec((NC, _BN, d1), lambda i: (0, i, 0)),
          pl.BlockSpec((NC, _BN, CW), lambda i: (0, i, 0)),
          pl.BlockSpec((_BN, d1), lambda i: (i, 0)),
          pl.BlockSpec((1, d1), lambda i: (0, 0)),
          pl.BlockSpec((d1, d2), lambda i: (0, 0)),
          pl.BlockSpec((d1, d2), lambda i: (0, 0)),
      ],
      out_specs=[
          pl.BlockSpec((_BN, d2), lambda i: (i, 0)),
          pl.BlockSpec((_BN, d2), lambda i: (i, 0)),
      ],
      out_shape=[
          jax.ShapeDtypeStruct((N, d2), jnp.float32),
          jax.ShapeDtypeStruct((N, d2), jnp.float32),
      ],
  )(s1, cnt, r1, b1, w2l_t, w2r_t)


def _tc_final(s2, cnt, r2, b2, wc_t, bc):
  d2 = r2.shape[1]
  c = wc_t.shape[1]

  def body(s_ref, c_ref, r_ref, b_ref, wc_ref, bc_ref, h_ref, z_ref):
    sv = s_ref[...]
    cv = c_ref[...]
    s = sv[0] + sv[1]
    cnt_col = cv[0, :, 0:1] + cv[1, :, 0:1]
    mean = s / jnp.maximum(cnt_col, 1.0)
    h2 = jnp.maximum(mean + b_ref[...] + r_ref[...], 0.0)
    h_ref[...] = h2
    z_ref[...] = (jnp.dot(h2, wc_ref[...], preferred_element_type=jnp.float32)
                  + bc_ref[...])

  return pl.pallas_call(
      body,
      grid=(N // _BN,),
      in_specs=[
          pl.BlockSpec((NC, _BN, d2), lambda i: (0, i, 0)),
          pl.BlockSpec((NC, _BN, CW), lambda i: (0, i, 0)),
          pl.BlockSpec((_BN, d2), lambda i: (i, 0)),
          pl.BlockSpec((1, d2), lambda i: (0, 0)),
          pl.BlockSpec((d2, c), lambda i: (0, 0)),
          pl.BlockSpec((1, c), lambda i: (0, 0)),
      ],
      out_specs=[
          pl.BlockSpec((_BN, d2), lambda i: (i, 0)),
          pl.BlockSpec((_BN, c), lambda i: (i, 0)),
      ],
      out_shape=[
          jax.ShapeDtypeStruct((N, d2), jnp.float32),
          jax.ShapeDtypeStruct((N, c), jnp.float32),
      ],
  )(s2, cnt, r2, b2, wc_t, bc)


def kernel(x, edge_index, W1_l, b1_l, W1_r, W2_l, b2_l, W2_r, Wc, bc):
  src = edge_index[0]
  dst = edge_index[1]
  h1_dim = W1_l.shape[0]
  h2_dim = W2_l.shape[0]

  p1, r1 = _tc_proj(x, W1_l.T, W1_r.T)
  s1, cnt = _seg_sum_sc(p1, src, dst, h1_dim, with_counts=True)
  p2, r2 = _tc_mid(s1, cnt, r1, b1_l.reshape(1, -1), W2_l.T, W2_r.T)
  s2 = _seg_sum_sc(p2, src, dst, h2_dim, with_counts=False)[0]
  h2, z = _tc_final(s2, cnt, r2, b2_l.reshape(1, -1), Wc.T,
                    bc.reshape(1, -1))
  return (h2, z)


# speedup vs baseline: 18.7800x; 2.9591x over previous
import functools

import jax
import jax.numpy as jnp
from jax import lax
from jax.experimental import pallas as pl
from jax.experimental.pallas import tpu as pltpu
from jax.experimental.pallas import tpu_sc as plsc

N = 10000
E = 320000
NC = 2
NS = 16
NW = NC * NS
E_PER_W = E // NW
CHUNK = 80
N_CHUNKS = E_PER_W // CHUNK
N_PAD = 10240
ROWS_PER_TILE = N_PAD // NS
ZROWS = 128
CW = 16


NBUF = 5
N_OUTER = N_CHUNKS // NBUF


def _seg_sum_sc(p, src2d, dst2d, d, with_counts):
  mesh = plsc.VectorSubcoreMesh(
      core_axis_name="c", subcore_axis_name="s", num_cores=NC,
      num_subcores=NS)

  out_type = [jax.ShapeDtypeStruct((NC, N_PAD, d), jnp.float32)]
  if with_counts:
    out_type.append(jax.ShapeDtypeStruct((NC, N_PAD, CW), jnp.float32))

  scratch = [
      pltpu.VMEM((N_CHUNKS, CHUNK), jnp.int32),
      pltpu.VMEM((N_CHUNKS, CHUNK), jnp.int32),
      [pltpu.VMEM((CHUNK, d), jnp.float32) for _ in range(NBUF)],
      [pltpu.SemaphoreType.DMA for _ in range(NBUF)],
      pltpu.VMEM((ZROWS, d), jnp.float32),
      pltpu.VMEM_SHARED((N_PAD, d), jnp.float32),
      pltpu.SemaphoreType.DMA,
  ]
  if with_counts:
    scratch += [
        pltpu.VMEM((CHUNK, CW), jnp.float32),
        pltpu.VMEM((ZROWS, CW), jnp.float32),
        pltpu.VMEM_SHARED((N_PAD, CW), jnp.float32),
    ]

  def body(p_hbm, src_hbm, dst_hbm, *rest):
    if with_counts:
      (sums_hbm, cnts_hbm, src_v, dst_v, bufs, gsems, zer_v, acc_sh, isem,
       ones_v, zerc_v, cnt_sh) = rest
    else:
      (sums_hbm, src_v, dst_v, bufs, gsems, zer_v, acc_sh, isem) = rest
    sid = lax.axis_index("s")
    cid = lax.axis_index("c")
    wid = sid * NC + cid

    pltpu.async_copy(src_hbm.at[pl.ds(wid * N_CHUNKS, N_CHUNKS)], src_v,
                     isem)
    pltpu.sync_copy(dst_hbm.at[pl.ds(wid * N_CHUNKS, N_CHUNKS)], dst_v)

    def init_row(i, _):
      for j in range(d // 16):
        zer_v[i, pl.ds(j * 16, 16)] = jnp.zeros((16,), jnp.float32)
      if with_counts:
        zerc_v[i, pl.ds(0, 16)] = jnp.zeros((16,), jnp.float32)
      return _
    lax.fori_loop(0, ZROWS, init_row, 0)
    if with_counts:
      def init_ones(i, _):
        ones_v[i, pl.ds(0, 16)] = jnp.ones((16,), jnp.float32)
        return _
      lax.fori_loop(0, CHUNK, init_ones, 0)

    base = sid * ROWS_PER_TILE
    for k in range(ROWS_PER_TILE // ZROWS):
      pltpu.sync_copy(zer_v, acc_sh.at[pl.ds(base + k * ZROWS, ZROWS)])
      if with_counts:
        pltpu.sync_copy(zerc_v, cnt_sh.at[pl.ds(base + k * ZROWS, ZROWS)])
    pltpu.make_async_copy(
        src_hbm.at[pl.ds(wid * N_CHUNKS, N_CHUNKS)], src_v, isem).wait()
    plsc.subcore_barrier()

    for b in range(NBUF):
      pltpu.async_copy(p_hbm.at[src_v.at[b]], bufs[b], gsems[b])

    def outer(g, _):
      for b in range(NBUF):
        i = g * NBUF + b
        pltpu.make_async_copy(p_hbm.at[src_v.at[0]], bufs[b],
                              gsems[b]).wait()
        pltpu.sync_copy(bufs[b], acc_sh.at[dst_v.at[i]], add=True)
        if with_counts:
          pltpu.sync_copy(ones_v, cnt_sh.at[dst_v.at[i]], add=True)

        @pl.when(i + NBUF < N_CHUNKS)
        def _start():
          pltpu.async_copy(p_hbm.at[src_v.at[i + NBUF]], bufs[b], gsems[b])
      return _
    lax.fori_loop(0, N_OUTER, outer, 0)

    plsc.subcore_barrier()
    pltpu.sync_copy(acc_sh.at[pl.ds(base, ROWS_PER_TILE)],
                    sums_hbm.at[cid, pl.ds(base, ROWS_PER_TILE)])
    if with_counts:
      pltpu.sync_copy(cnt_sh.at[pl.ds(base, ROWS_PER_TILE)],
                      cnts_hbm.at[cid, pl.ds(base, ROWS_PER_TILE)])

  fn = pl.kernel(
      body, out_type=out_type, mesh=mesh, scratch_types=scratch,
      compiler_params=pltpu.CompilerParams(use_tc_tiling_on_sc=False))
  return fn(p, src2d, dst2d)


_BN = 2000


def _tc_proj(x, wl_t, wr_t):
  d_in, d_out = wl_t.shape

  def body(x_ref, wl_ref, wr_ref, p_ref, r_ref):
    xb = x_ref[...]
    p_ref[...] = jnp.dot(xb, wl_ref[...], preferred_element_type=jnp.float32)
    r_ref[...] = jnp.dot(xb, wr_ref[...], preferred_element_type=jnp.float32)

  return pl.pallas_call(
      body,
      grid=(N // _BN,),
      in_specs=[
          pl.BlockSpec((_BN, d_in), lambda i: (i, 0)),
          pl.BlockSpec((d_in, d_out), lambda i: (0, 0)),
          pl.BlockSpec((d_in, d_out), lambda i: (0, 0)),
      ],
      out_specs=[
          pl.BlockSpec((_BN, d_out), lambda i: (i, 0)),
          pl.BlockSpec((_BN, d_out), lambda i: (i, 0)),
      ],
      out_shape=[
          jax.ShapeDtypeStruct((N, d_out), jnp.float32),
          jax.ShapeDtypeStruct((N, d_out), jnp.float32),
      ],
  )(x, wl_t, wr_t)


def _tc_mid(s1, cnt, r1, b1, w2l_t, w2r_t):
  d1 = r1.shape[1]
  d2 = w2l_t.shape[1]

  def body(s_ref, c_ref, r_ref, b_ref, wl_ref, wr_ref, p_ref, rr_ref):
    sv = s_ref[...]
    cv = c_ref[...]
    s = sv[0] + sv[1]
    cnt_col = cv[0, :, 0:1] + cv[1, :, 0:1]
    mean = s / jnp.maximum(cnt_col, 1.0)
    h1 = jnp.maximum(mean + b_ref[...] + r_ref[...], 0.0)
    p_ref[...] = jnp.dot(h1, wl_ref[...], preferred_element_type=jnp.float32)
    rr_ref[...] = jnp.dot(h1, wr_ref[...], preferred_element_type=jnp.float32)

  return pl.pallas_call(
      body,
      grid=(N // _BN,),
      in_specs=[
          pl.BlockSpec((NC, _BN, d1), lambda i: (0, i, 0)),
          pl.BlockSpec((NC, _BN, CW), lambda i: (0, i, 0)),
          pl.BlockSpec((_BN, d1), lambda i: (i, 0)),
          pl.BlockSpec((1, d1), lambda i: (0, 0)),
          pl.BlockSpec((d1, d2), lambda i: (0, 0)),
          pl.BlockSpec((d1, d2), lambda i: (0, 0)),
      ],
      out_specs=[
          pl.BlockSpec((_BN, d2), lambda i: (i, 0)),
          pl.BlockSpec((_BN, d2), lambda i: (i, 0)),
      ],
      out_shape=[
          jax.ShapeDtypeStruct((N, d2), jnp.float32),
          jax.ShapeDtypeStruct((N, d2), jnp.float32),
      ],
  )(s1, cnt, r1, b1, w2l_t, w2r_t)


def _tc_final(s2, cnt, r2, b2, wc_t, bc):
  d2 = r2.shape[1]
  c = wc_t.shape[1]

  def body(s_ref, c_ref, r_ref, b_ref, wc_ref, bc_ref, h_ref, z_ref):
    sv = s_ref[...]
    cv = c_ref[...]
    s = sv[0] + sv[1]
    cnt_col = cv[0, :, 0:1] + cv[1, :, 0:1]
    mean = s / jnp.maximum(cnt_col, 1.0)
    h2 = jnp.maximum(mean + b_ref[...] + r_ref[...], 0.0)
    h_ref[...] = h2
    z_ref[...] = (jnp.dot(h2, wc_ref[...], preferred_element_type=jnp.float32)
                  + bc_ref[...])

  return pl.pallas_call(
      body,
      grid=(N // _BN,),
      in_specs=[
          pl.BlockSpec((NC, _BN, d2), lambda i: (0, i, 0)),
          pl.BlockSpec((NC, _BN, CW), lambda i: (0, i, 0)),
          pl.BlockSpec((_BN, d2), lambda i: (i, 0)),
          pl.BlockSpec((1, d2), lambda i: (0, 0)),
          pl.BlockSpec((d2, c), lambda i: (0, 0)),
          pl.BlockSpec((1, c), lambda i: (0, 0)),
      ],
      out_specs=[
          pl.BlockSpec((_BN, d2), lambda i: (i, 0)),
          pl.BlockSpec((_BN, c), lambda i: (i, 0)),
      ],
      out_shape=[
          jax.ShapeDtypeStruct((N, d2), jnp.float32),
          jax.ShapeDtypeStruct((N, c), jnp.float32),
      ],
  )(s2, cnt, r2, b2, wc_t, bc)


def kernel(x, edge_index, W1_l, b1_l, W1_r, W2_l, b2_l, W2_r, Wc, bc):
  src = edge_index[0].reshape(E // CHUNK, CHUNK)
  dst = edge_index[1].reshape(E // CHUNK, CHUNK)
  h1_dim = W1_l.shape[0]
  h2_dim = W2_l.shape[0]

  p1, r1 = _tc_proj(x, W1_l.T, W1_r.T)
  s1, cnt = _seg_sum_sc(p1, src, dst, h1_dim, with_counts=True)
  p2, r2 = _tc_mid(s1, cnt, r1, b1_l.reshape(1, -1), W2_l.T, W2_r.T)
  s2 = _seg_sum_sc(p2, src, dst, h2_dim, with_counts=False)[0]
  h2, z = _tc_final(s2, cnt, r2, b2_l.reshape(1, -1), Wc.T,
                    bc.reshape(1, -1))
  return (h2, z)


# speedup vs baseline: 19.0868x; 1.0163x over previous
import functools

import jax
import jax.numpy as jnp
from jax import lax
from jax.experimental import pallas as pl
from jax.experimental.pallas import tpu as pltpu
from jax.experimental.pallas import tpu_sc as plsc

N = 10000
E = 320000
NC = 2
NS = 16
NW = NC * NS
E_PER_W = E // NW
CHUNK = 80
N_CHUNKS = E_PER_W // CHUNK
N_PAD = 10240
ROWS_PER_TILE = N_PAD // NS
CW = 16


ZROWS = 64


def _seg_sum_sc(p, src2d, dst2d, d, with_counts, nbuf):
  mesh = plsc.VectorSubcoreMesh(
      core_axis_name="c", subcore_axis_name="s", num_cores=NC,
      num_subcores=NS)

  out_type = [jax.ShapeDtypeStruct((NC, N_PAD, d), jnp.float32)]
  if with_counts:
    out_type.append(jax.ShapeDtypeStruct((NC, N_PAD, CW), jnp.float32))

  scratch = [
      pltpu.VMEM((N_CHUNKS, CHUNK), jnp.int32),
      pltpu.VMEM((N_CHUNKS, CHUNK), jnp.int32),
      [pltpu.VMEM((CHUNK, d), jnp.float32) for _ in range(nbuf)],
      [pltpu.SemaphoreType.DMA for _ in range(nbuf)],
      pltpu.VMEM((ZROWS, d), jnp.float32),
      pltpu.VMEM_SHARED((N_PAD, d), jnp.float32),
      pltpu.SemaphoreType.DMA,
  ]
  if with_counts:
    scratch += [
        pltpu.VMEM((CHUNK, CW), jnp.float32),
        pltpu.VMEM((ZROWS, CW), jnp.float32),
        pltpu.VMEM_SHARED((N_PAD, CW), jnp.float32),
    ]

  def body(p_hbm, src_hbm, dst_hbm, *rest):
    if with_counts:
      (sums_hbm, cnts_hbm, src_v, dst_v, bufs, gsems, zer_v, acc_sh, isem,
       ones_v, zerc_v, cnt_sh) = rest
    else:
      (sums_hbm, src_v, dst_v, bufs, gsems, zer_v, acc_sh, isem) = rest
    sid = lax.axis_index("s")
    cid = lax.axis_index("c")
    wid = sid * NC + cid

    pltpu.async_copy(src_hbm.at[pl.ds(wid * N_CHUNKS, N_CHUNKS)], src_v,
                     isem)
    pltpu.sync_copy(dst_hbm.at[pl.ds(wid * N_CHUNKS, N_CHUNKS)], dst_v)

    def init_row(i, _):
      for j in range(d // 16):
        zer_v[i, pl.ds(j * 16, 16)] = jnp.zeros((16,), jnp.float32)
      if with_counts:
        zerc_v[i, pl.ds(0, 16)] = jnp.zeros((16,), jnp.float32)
      return _
    lax.fori_loop(0, ZROWS, init_row, 0)
    if with_counts:
      def init_ones(i, _):
        ones_v[i, pl.ds(0, 16)] = jnp.ones((16,), jnp.float32)
        return _
      lax.fori_loop(0, CHUNK, init_ones, 0)

    base = sid * ROWS_PER_TILE
    for k in range(ROWS_PER_TILE // ZROWS):
      pltpu.sync_copy(zer_v, acc_sh.at[pl.ds(base + k * ZROWS, ZROWS)])
      if with_counts:
        pltpu.sync_copy(zerc_v, cnt_sh.at[pl.ds(base + k * ZROWS, ZROWS)])
    pltpu.make_async_copy(
        src_hbm.at[pl.ds(wid * N_CHUNKS, N_CHUNKS)], src_v, isem).wait()
    plsc.subcore_barrier()

    n_outer = N_CHUNKS // nbuf
    n_tail = N_CHUNKS - n_outer * nbuf
    for b in range(nbuf):
      pltpu.async_copy(p_hbm.at[src_v.at[b]], bufs[b], gsems[b])

    def _drain(i, b):
      pltpu.make_async_copy(p_hbm.at[src_v.at[0]], bufs[b],
                            gsems[b]).wait()
      pltpu.sync_copy(bufs[b], acc_sh.at[dst_v.at[i]], add=True)
      if with_counts:
        pltpu.sync_copy(ones_v, cnt_sh.at[dst_v.at[i]], add=True)

    def outer(g, _):
      for b in range(nbuf):
        i = g * nbuf + b
        _drain(i, b)

        @pl.when(i + nbuf < N_CHUNKS)
        def _start():
          pltpu.async_copy(p_hbm.at[src_v.at[i + nbuf]], bufs[b], gsems[b])
      return _
    lax.fori_loop(0, n_outer, outer, 0)
    for b in range(n_tail):
      _drain(n_outer * nbuf + b, b)

    plsc.subcore_barrier()
    pltpu.sync_copy(acc_sh.at[pl.ds(base, ROWS_PER_TILE)],
                    sums_hbm.at[cid, pl.ds(base, ROWS_PER_TILE)])
    if with_counts:
      pltpu.sync_copy(cnt_sh.at[pl.ds(base, ROWS_PER_TILE)],
                      cnts_hbm.at[cid, pl.ds(base, ROWS_PER_TILE)])

  fn = pl.kernel(
      body, out_type=out_type, mesh=mesh, scratch_types=scratch,
      compiler_params=pltpu.CompilerParams(use_tc_tiling_on_sc=False))
  return fn(p, src2d, dst2d)


_BN = 2000


def _tc_proj(x, wl_t, wr_t):
  d_in, d_out = wl_t.shape

  def body(x_ref, wl_ref, wr_ref, p_ref, r_ref):
    xb = x_ref[...]
    p_ref[...] = jnp.dot(xb, wl_ref[...], preferred_element_type=jnp.float32)
    r_ref[...] = jnp.dot(xb, wr_ref[...], preferred_element_type=jnp.float32)

  return pl.pallas_call(
      body,
      grid=(N // _BN,),
      in_specs=[
          pl.BlockSpec((_BN, d_in), lambda i: (i, 0)),
          pl.BlockSpec((d_in, d_out), lambda i: (0, 0)),
          pl.BlockSpec((d_in, d_out), lambda i: (0, 0)),
      ],
      out_specs=[
          pl.BlockSpec((_BN, d_out), lambda i: (i, 0)),
          pl.BlockSpec((_BN, d_out), lambda i: (i, 0)),
      ],
      out_shape=[
          jax.ShapeDtypeStruct((N, d_out), jnp.float32),
          jax.ShapeDtypeStruct((N, d_out), jnp.float32),
      ],
  )(x, wl_t, wr_t)


def _tc_mid(s1, cnt, r1, b1, w2l_t, w2r_t):
  d1 = r1.shape[1]
  d2 = w2l_t.shape[1]

  def body(s_ref, c_ref, r_ref, b_ref, wl_ref, wr_ref, p_ref, rr_ref):
    sv = s_ref[...]
    cv = c_ref[...]
    s = sv[0] + sv[1]
    cnt_col = cv[0, :, 0:1] + cv[1, :, 0:1]
    mean = s / jnp.maximum(cnt_col, 1.0)
    h1 = jnp.maximum(mean + b_ref[...] + r_ref[...], 0.0)
    p_ref[...] = jnp.dot(h1, wl_ref[...], preferred_element_type=jnp.float32)
    rr_ref[...] = jnp.dot(h1, wr_ref[...], preferred_element_type=jnp.float32)

  return pl.pallas_call(
      body,
      grid=(N // _BN,),
      in_specs=[
          pl.BlockSpec((NC, _BN, d1), lambda i: (0, i, 0)),
          pl.BlockSpec((NC, _BN, CW), lambda i: (0, i, 0)),
          pl.BlockSpec((_BN, d1), lambda i: (i, 0)),
          pl.BlockSpec((1, d1), lambda i: (0, 0)),
          pl.BlockSpec((d1, d2), lambda i: (0, 0)),
          pl.BlockSpec((d1, d2), lambda i: (0, 0)),
      ],
      out_specs=[
          pl.BlockSpec((_BN, d2), lambda i: (i, 0)),
          pl.BlockSpec((_BN, d2), lambda i: (i, 0)),
      ],
      out_shape=[
          jax.ShapeDtypeStruct((N, d2), jnp.float32),
          jax.ShapeDtypeStruct((N, d2), jnp.float32),
      ],
  )(s1, cnt, r1, b1, w2l_t, w2r_t)


def _tc_final(s2, cnt, r2, b2, wc_t, bc):
  d2 = r2.shape[1]
  c = wc_t.shape[1]

  def body(s_ref, c_ref, r_ref, b_ref, wc_ref, bc_ref, h_ref, z_ref):
    sv = s_ref[...]
    cv = c_ref[...]
    s = sv[0] + sv[1]
    cnt_col = cv[0, :, 0:1] + cv[1, :, 0:1]
    mean = s / jnp.maximum(cnt_col, 1.0)
    h2 = jnp.maximum(mean + b_ref[...] + r_ref[...], 0.0)
    h_ref[...] = h2
    z_ref[...] = (jnp.dot(h2, wc_ref[...], preferred_element_type=jnp.float32)
                  + bc_ref[...])

  return pl.pallas_call(
      body,
      grid=(N // _BN,),
      in_specs=[
          pl.BlockSpec((NC, _BN, d2), lambda i: (0, i, 0)),
          pl.BlockSpec((NC, _BN, CW), lambda i: (0, i, 0)),
          pl.BlockSpec((_BN, d2), lambda i: (i, 0)),
          pl.BlockSpec((1, d2), lambda i: (0, 0)),
          pl.BlockSpec((d2, c), lambda i: (0, 0)),
          pl.BlockSpec((1, c), lambda i: (0, 0)),
      ],
      out_specs=[
          pl.BlockSpec((_BN, d2), lambda i: (i, 0)),
          pl.BlockSpec((_BN, c), lambda i: (i, 0)),
      ],
      out_shape=[
          jax.ShapeDtypeStruct((N, d2), jnp.float32),
          jax.ShapeDtypeStruct((N, c), jnp.float32),
      ],
  )(s2, cnt, r2, b2, wc_t, bc)


def kernel(x, edge_index, W1_l, b1_l, W1_r, W2_l, b2_l, W2_r, Wc, bc):
  src = edge_index[0].reshape(E // CHUNK, CHUNK)
  dst = edge_index[1].reshape(E // CHUNK, CHUNK)
  h1_dim = W1_l.shape[0]
  h2_dim = W2_l.shape[0]

  p1, r1 = _tc_proj(x, W1_l.T, W1_r.T)
  s1, cnt = _seg_sum_sc(p1, src, dst, h1_dim, with_counts=True, nbuf=8)
  p2, r2 = _tc_mid(s1, cnt, r1, b1_l.reshape(1, -1), W2_l.T, W2_r.T)
  s2 = _seg_sum_sc(p2, src, dst, h2_dim, with_counts=False, nbuf=12)[0]
  h2, z = _tc_final(s2, cnt, r2, b2_l.reshape(1, -1), Wc.T,
                    bc.reshape(1, -1))
  return (h2, z)


# speedup vs baseline: 25.2334x; 1.3220x over previous
import jax
import jax.numpy as jnp
from jax import lax
from jax.experimental import pallas as pl
from jax.experimental.pallas import tpu as pltpu
from jax.experimental.pallas import tpu_sc as plsc

N = 10000
E = 320000
NC = 2
NS = 16
NW = NC * NS
CHUNK = 128
NCHK = E // CHUNK
BASE_CHK = NCHK // NW
EXTRA_W = NCHK - BASE_CHK * NW
N_PAD = 10240
ROWS_PER_TILE = N_PAD // NS
ZROWS = 64
CW = 16


def _seg_sum_sc(p, ei3, d, mul, with_counts, nbuf):
  mesh = plsc.VectorSubcoreMesh(
      core_axis_name="c", subcore_axis_name="s", num_cores=NC,
      num_subcores=NS)

  out_type = [jax.ShapeDtypeStruct((NC, N_PAD, 128), jnp.float32)]

  scratch = [
      pltpu.VMEM((BASE_CHK + 1, 2, CHUNK), jnp.int32),
      pltpu.VMEM((nbuf, CHUNK), jnp.int32),
      [pltpu.VMEM((CHUNK, d), jnp.float32) for _ in range(nbuf)],
      [pltpu.SemaphoreType.DMA for _ in range(nbuf)],
      pltpu.VMEM((ZROWS, d), jnp.float32),
      pltpu.VMEM_SHARED((N_PAD, d), jnp.float32),
      pltpu.SemaphoreType.DMA,
  ]
  if with_counts:
    scratch += [
        pltpu.VMEM((CHUNK, CW), jnp.float32),
        pltpu.VMEM((ZROWS, CW), jnp.float32),
        pltpu.VMEM_SHARED((N_PAD, CW), jnp.float32),
    ]

  def body(p_hbm, ei_hbm, *rest):
    if with_counts:
      (out_hbm, ei_v, idx_v, bufs, gsems, zer_v, acc_sh, isem,
       ones_v, zerc_v, cnt_sh) = rest
    else:
      (out_hbm, ei_v, idx_v, bufs, gsems, zer_v, acc_sh, isem) = rest
    sid = lax.axis_index("s")
    cid = lax.axis_index("c")
    wid = sid * NC + cid
    start_w = wid * BASE_CHK + jnp.minimum(wid, EXTRA_W)
    has_extra = wid < EXTRA_W
    my_chunks = jnp.where(has_extra, BASE_CHK + 1, BASE_CHK)

    pltpu.async_copy(ei_hbm.at[pl.ds(start_w, BASE_CHK)],
                     ei_v.at[pl.ds(0, BASE_CHK)], isem)

    @pl.when(has_extra)
    def _extra_row():
      pltpu.async_copy(ei_hbm.at[pl.ds(start_w + BASE_CHK, 1)],
                       ei_v.at[pl.ds(BASE_CHK, 1)], isem)

    def init_row(i, _):
      for j in range(d // 16):
        zer_v[i, pl.ds(j * 16, 16)] = jnp.zeros((16,), jnp.float32)
      if with_counts:
        zerc_v[i, pl.ds(0, 16)] = jnp.zeros((16,), jnp.float32)
      return _
    lax.fori_loop(0, ZROWS, init_row, 0)
    if with_counts:
      def init_ones(i, _):
        ones_v[i, pl.ds(0, 16)] = jnp.ones((16,), jnp.float32)
        return _
      lax.fori_loop(0, CHUNK, init_ones, 0)

    base = sid * ROWS_PER_TILE
    for k in range(ROWS_PER_TILE // ZROWS):
      pltpu.sync_copy(zer_v, acc_sh.at[pl.ds(base + k * ZROWS, ZROWS)])
      if with_counts:
        pltpu.sync_copy(zerc_v, cnt_sh.at[pl.ds(base + k * ZROWS, ZROWS)])

    pltpu.make_async_copy(ei_hbm.at[pl.ds(start_w, BASE_CHK)],
                          ei_v.at[pl.ds(0, BASE_CHK)], isem).wait()

    @pl.when(has_extra)
    def _extra_wait():
      pltpu.make_async_copy(ei_hbm.at[pl.ds(start_w + BASE_CHK, 1)],
                            ei_v.at[pl.ds(BASE_CHK, 1)], isem).wait()

    plsc.subcore_barrier()

    def _start_gather(j, b):
      for t in range(CHUNK // 16):
        idx_v[b, pl.ds(t * 16, 16)] = (
            ei_v[j, 0, pl.ds(t * 16, 16)] * mul)
      pltpu.async_copy(p_hbm.at[idx_v.at[b]], bufs[b], gsems[b])

    for b in range(nbuf):
      _start_gather(b, b)

    def _drain(i, b):
      pltpu.make_async_copy(p_hbm.at[idx_v.at[b]], bufs[b],
                            gsems[b]).wait()
      pltpu.sync_copy(bufs[b], acc_sh.at[ei_v.at[i, 1]], add=True)
      if with_counts:
        pltpu.sync_copy(ones_v, cnt_sh.at[ei_v.at[i, 1]], add=True)

    n_outer = BASE_CHK // nbuf
    n_tail = BASE_CHK - n_outer * nbuf

    def outer(g, _):
      for b in range(nbuf):
        i = g * nbuf + b
        _drain(i, b)

        @pl.when(i + nbuf < my_chunks)
        def _start():
          _start_gather(i + nbuf, b)
      return _
    lax.fori_loop(0, n_outer, outer, 0)
    for b in range(n_tail):
      _drain(n_outer * nbuf + b, (n_outer * nbuf + b) % nbuf)

    @pl.when(has_extra)
    def _extra_chunk():
      _drain(BASE_CHK, BASE_CHK % nbuf)

    plsc.subcore_barrier()
    pltpu.sync_copy(acc_sh.at[pl.ds(base, ROWS_PER_TILE)],
                    out_hbm.at[cid, pl.ds(base, ROWS_PER_TILE),
                               pl.ds(0, d)])
    if with_counts:
      pltpu.sync_copy(cnt_sh.at[pl.ds(base, ROWS_PER_TILE)],
                      out_hbm.at[cid, pl.ds(base, ROWS_PER_TILE),
                                 pl.ds(64, CW)])

  fn = pl.kernel(
      body, out_type=out_type, mesh=mesh, scratch_types=scratch,
      compiler_params=pltpu.CompilerParams(use_tc_tiling_on_sc=False))
  return fn(p, ei3)


_BN = 2000


def _dot_t(a, w):
  return lax.dot_general(a, w, (((1,), (1,)), ((), ())),
                         preferred_element_type=jnp.float32)


def _tc_proj(x, w1l, w1r):
  def body(x_ref, wl_ref, wr_ref, pp_ref):
    xb = x_ref[...]
    pp_ref[...] = jnp.concatenate(
        [_dot_t(xb, wl_ref[...]), _dot_t(xb, wr_ref[...])], axis=1)

  return pl.pallas_call(
      body,
      grid=(N // _BN,),
      in_specs=[
          pl.BlockSpec((_BN, 128), lambda i: (i, 0)),
          pl.BlockSpec((64, 128), lambda i: (0, 0)),
          pl.BlockSpec((64, 128), lambda i: (0, 0)),
      ],
      out_specs=pl.BlockSpec((_BN, 128), lambda i: (i, 0)),
      out_shape=jax.ShapeDtypeStruct((N, 128), jnp.float32),
  )(x, w1l, w1r)


def _tc_mid(s1fat, pfat1, b1, w2l, w2r):
  def body(s_ref, p_ref, b_ref, wl_ref, wr_ref, o_ref):
    sf = s_ref[...]
    s = sf[0, :N, 0:64] + sf[1, :N, 0:64]
    cnt = sf[0, :N, 64:65] + sf[1, :N, 64:65]
    pf = p_ref[...]
    r1 = pf[:, 64:128]
    mean = s / jnp.maximum(cnt, 1.0)
    h1 = jnp.maximum(mean + b_ref[...] + r1, 0.0)
    p2 = _dot_t(h1, wl_ref[...])
    r2 = _dot_t(h1, wr_ref[...])
    cnt32 = jnp.broadcast_to(cnt, (N, 32))
    o_ref[...] = jnp.concatenate([p2, r2, cnt32, p2], axis=1)

  return pl.pallas_call(
      body,
      out_shape=jax.ShapeDtypeStruct((N, 128), jnp.float32),
  )(s1fat, pfat1, b1, w2l, w2r)


def _tc_final(s2fat, fat2, b2, wc, bc):
  def body(s_ref, f_ref, b_ref, wc_ref, bc_ref, h_ref, z_ref):
    sf = s_ref[...]
    s = sf[0, :N, 0:32] + sf[1, :N, 0:32]
    f2 = f_ref[...]
    r2 = f2[:, 32:64]
    cnt = f2[:, 64:65]
    mean = s / jnp.maximum(cnt, 1.0)
    h2 = jnp.maximum(mean + b_ref[...] + r2, 0.0)
    h_ref[...] = h2
    z_ref[...] = _dot_t(h2, wc_ref[...]) + bc_ref[...]

  return pl.pallas_call(
      body,
      out_shape=[
          jax.ShapeDtypeStruct((N, 32), jnp.float32),
          jax.ShapeDtypeStruct((N, 4), jnp.float32),
      ],
  )(s2fat, fat2, b2, wc, bc)


def kernel(x, edge_index, W1_l, b1_l, W1_r, W2_l, b2_l, W2_r, Wc, bc):
  ei3 = edge_index.reshape(2, NCHK, CHUNK).transpose(1, 0, 2)

  pfat1 = _tc_proj(x, W1_l, W1_r)
  s1fat = _seg_sum_sc(pfat1.reshape(2 * N, 64), ei3, 64, mul=2,
                      with_counts=True, nbuf=5)[0]
  fat2 = _tc_mid(s1fat, pfat1, b1_l.reshape(1, -1), W2_l, W2_r)
  s2fat = _seg_sum_sc(fat2.reshape(4 * N, 32), ei3, 32, mul=4,
                      with_counts=False, nbuf=8)[0]
  h2, z = _tc_final(s2fat, fat2, b2_l.reshape(1, -1), Wc,
                    bc.reshape(1, -1))
  return (h2, z)


# speedup vs baseline: 26.7202x; 1.0589x over previous
import jax
import jax.numpy as jnp
from jax import lax
from jax.experimental import pallas as pl
from jax.experimental.pallas import tpu as pltpu
from jax.experimental.pallas import tpu_sc as plsc

N = 10000
E = 320000
NC = 2
NS = 16
NW = NC * NS
CHUNK = 128
NCHK = E // CHUNK
BASE_CHK = NCHK // NW
EXTRA_W = NCHK - BASE_CHK * NW
N_PAD = 10240
ROWS_PER_TILE = N_PAD // NS
ZROWS = 64
CW = 16


def _seg_sum_sc(p, ei3, d, mul, with_counts, nbuf):
  mesh = plsc.VectorSubcoreMesh(
      core_axis_name="c", subcore_axis_name="s", num_cores=NC,
      num_subcores=NS)

  out_type = [jax.ShapeDtypeStruct((NC, N_PAD, 128), jnp.float32)]

  scratch = [
      pltpu.VMEM((BASE_CHK + 1, 2, CHUNK), jnp.int32),
      [pltpu.VMEM((CHUNK, d), jnp.float32) for _ in range(nbuf)],
      [pltpu.SemaphoreType.DMA for _ in range(nbuf)],
      pltpu.VMEM((ZROWS, d), jnp.float32),
      pltpu.VMEM_SHARED((N_PAD, d), jnp.float32),
      pltpu.SemaphoreType.DMA,
  ]
  if with_counts:
    scratch += [
        pltpu.VMEM((CHUNK, CW), jnp.float32),
        pltpu.VMEM((ZROWS, CW), jnp.float32),
        pltpu.VMEM_SHARED((N_PAD, CW), jnp.float32),
    ]

  def body(p_hbm, ei_hbm, *rest):
    if with_counts:
      (out_hbm, ei_v, bufs, gsems, zer_v, acc_sh, isem,
       ones_v, zerc_v, cnt_sh) = rest
    else:
      (out_hbm, ei_v, bufs, gsems, zer_v, acc_sh, isem) = rest
    sid = lax.axis_index("s")
    cid = lax.axis_index("c")
    wid = sid * NC + cid
    start_w = wid * BASE_CHK + jnp.minimum(wid, EXTRA_W)
    has_extra = wid < EXTRA_W
    my_chunks = jnp.where(has_extra, BASE_CHK + 1, BASE_CHK)

    pltpu.async_copy(ei_hbm.at[pl.ds(start_w, BASE_CHK)],
                     ei_v.at[pl.ds(0, BASE_CHK)], isem)

    @pl.when(has_extra)
    def _extra_row():
      pltpu.async_copy(ei_hbm.at[pl.ds(start_w + BASE_CHK, 1)],
                       ei_v.at[pl.ds(BASE_CHK, 1)], isem)

    def init_row(i, _):
      for j in range(d // 16):
        zer_v[i, pl.ds(j * 16, 16)] = jnp.zeros((16,), jnp.float32)
      if with_counts:
        zerc_v[i, pl.ds(0, 16)] = jnp.zeros((16,), jnp.float32)
      return _
    lax.fori_loop(0, ZROWS, init_row, 0)
    if with_counts:
      def init_ones(i, _):
        ones_v[i, pl.ds(0, 16)] = jnp.ones((16,), jnp.float32)
        return _
      lax.fori_loop(0, CHUNK, init_ones, 0)

    base = sid * ROWS_PER_TILE
    for k in range(ROWS_PER_TILE // ZROWS):
      pltpu.sync_copy(zer_v, acc_sh.at[pl.ds(base + k * ZROWS, ZROWS)])
      if with_counts:
        pltpu.sync_copy(zerc_v, cnt_sh.at[pl.ds(base + k * ZROWS, ZROWS)])

    pltpu.make_async_copy(ei_hbm.at[pl.ds(start_w, BASE_CHK)],
                          ei_v.at[pl.ds(0, BASE_CHK)], isem).wait()

    @pl.when(has_extra)
    def _extra_wait():
      pltpu.make_async_copy(ei_hbm.at[pl.ds(start_w + BASE_CHK, 1)],
                            ei_v.at[pl.ds(BASE_CHK, 1)], isem).wait()

    def scale_row(j, _):
      for t in range(CHUNK // 16):
        ei_v[j, 0, pl.ds(t * 16, 16)] = (
            ei_v[j, 0, pl.ds(t * 16, 16)] * mul)
      return _
    lax.fori_loop(0, my_chunks, scale_row, 0)

    plsc.subcore_barrier()

    def _start_gather(j, b):
      pltpu.async_copy(p_hbm.at[ei_v.at[j, 0]], bufs[b], gsems[b])

    for b in range(nbuf):
      _start_gather(b, b)

    def _drain(i, b):
      pltpu.make_async_copy(p_hbm.at[ei_v.at[0, 0]], bufs[b],
                            gsems[b]).wait()
      pltpu.sync_copy(bufs[b], acc_sh.at[ei_v.at[i, 1]], add=True)
      if with_counts:
        pltpu.sync_copy(ones_v, cnt_sh.at[ei_v.at[i, 1]], add=True)

    n_outer = BASE_CHK // nbuf
    n_tail = BASE_CHK - n_outer * nbuf

    def outer(g, _):
      for b in range(nbuf):
        i = g * nbuf + b
        _drain(i, b)

        @pl.when(i + nbuf < my_chunks)
        def _start():
          _start_gather(i + nbuf, b)
      return _
    lax.fori_loop(0, n_outer, outer, 0)
    for b in range(n_tail):
      _drain(n_outer * nbuf + b, (n_outer * nbuf + b) % nbuf)

    @pl.when(has_extra)
    def _extra_chunk():
      _drain(BASE_CHK, BASE_CHK % nbuf)

    plsc.subcore_barrier()
    pltpu.sync_copy(acc_sh.at[pl.ds(base, ROWS_PER_TILE)],
                    out_hbm.at[cid, pl.ds(base, ROWS_PER_TILE),
                               pl.ds(0, d)])
    if with_counts:
      pltpu.sync_copy(cnt_sh.at[pl.ds(base, ROWS_PER_TILE)],
                      out_hbm.at[cid, pl.ds(base, ROWS_PER_TILE),
                                 pl.ds(64, CW)])

  fn = pl.kernel(
      body, out_type=out_type, mesh=mesh, scratch_types=scratch,
      compiler_params=pltpu.CompilerParams(use_tc_tiling_on_sc=False))
  return fn(p, ei3)


_BN = 2000


def _dot_t(a, w):
  return lax.dot_general(a, w, (((1,), (1,)), ((), ())),
                         preferred_element_type=jnp.float32)


def _tc_proj(x, w1l, w1r):
  def body(x_ref, wl_ref, wr_ref, pp_ref):
    xb = x_ref[...]
    pp_ref[...] = jnp.concatenate(
        [_dot_t(xb, wl_ref[...]), _dot_t(xb, wr_ref[...])], axis=1)

  return pl.pallas_call(
      body,
      grid=(N // _BN,),
      in_specs=[
          pl.BlockSpec((_BN, 128), lambda i: (i, 0)),
          pl.BlockSpec((64, 128), lambda i: (0, 0)),
          pl.BlockSpec((64, 128), lambda i: (0, 0)),
      ],
      out_specs=pl.BlockSpec((_BN, 128), lambda i: (i, 0)),
      out_shape=jax.ShapeDtypeStruct((N, 128), jnp.float32),
  )(x, w1l, w1r)


def _tc_mid(s1fat, pfat1, b1, w2l, w2r):
  def body(s_ref, p_ref, b_ref, wl_ref, wr_ref, o_ref):
    sf = s_ref[...]
    s = sf[0, :N, 0:64] + sf[1, :N, 0:64]
    cnt = sf[0, :N, 64:65] + sf[1, :N, 64:65]
    pf = p_ref[...]
    r1 = pf[:, 64:128]
    mean = s / jnp.maximum(cnt, 1.0)
    h1 = jnp.maximum(mean + b_ref[...] + r1, 0.0)
    p2 = _dot_t(h1, wl_ref[...])
    r2 = _dot_t(h1, wr_ref[...])
    cnt32 = jnp.broadcast_to(cnt, (N, 32))
    o_ref[...] = jnp.concatenate([p2, r2, cnt32, p2], axis=1)

  return pl.pallas_call(
      body,
      out_shape=jax.ShapeDtypeStruct((N, 128), jnp.float32),
  )(s1fat, pfat1, b1, w2l, w2r)


def _tc_final(s2fat, fat2, b2, wc, bc):
  def body(s_ref, f_ref, b_ref, wc_ref, bc_ref, h_ref, z_ref):
    sf = s_ref[...]
    s = sf[0, :N, 0:32] + sf[1, :N, 0:32]
    f2 = f_ref[...]
    r2 = f2[:, 32:64]
    cnt = f2[:, 64:65]
    mean = s / jnp.maximum(cnt, 1.0)
    h2t = jnp.transpose(jnp.maximum(mean + b_ref[...] + r2, 0.0))
    h_ref[...] = h2t
    z_ref[...] = (jnp.dot(wc_ref[...], h2t,
                          preferred_element_type=jnp.float32)
                  + bc_ref[...])

  return pl.pallas_call(
      body,
      out_shape=[
          jax.ShapeDtypeStruct((32, N), jnp.float32),
          jax.ShapeDtypeStruct((4, N), jnp.float32),
      ],
  )(s2fat, fat2, b2, wc, bc)


def kernel(x, edge_index, W1_l, b1_l, W1_r, W2_l, b2_l, W2_r, Wc, bc):
  ei3 = edge_index.reshape(2, NCHK, CHUNK).transpose(1, 0, 2)

  pfat1 = _tc_proj(x, W1_l, W1_r)
  s1fat = _seg_sum_sc(pfat1.reshape(2 * N, 64), ei3, 64, mul=2,
                      with_counts=True, nbuf=5)[0]
  fat2 = _tc_mid(s1fat, pfat1, b1_l.reshape(1, -1), W2_l, W2_r)
  s2fat = _seg_sum_sc(fat2.reshape(4 * N, 32), ei3, 32, mul=4,
                      with_counts=False, nbuf=12)[0]
  h2t, zt = _tc_final(s2fat, fat2, b2_l.reshape(1, -1), Wc,
                      bc.reshape(-1, 1))
  return (h2t.T, zt.T)


# speedup vs baseline: 26.9616x; 1.0090x over previous
import jax
import jax.numpy as jnp
from jax import lax
from jax.experimental import pallas as pl
from jax.experimental.pallas import tpu as pltpu
from jax.experimental.pallas import tpu_sc as plsc

N = 10000
E = 320000
NC = 2
NS = 16
NW = NC * NS
CHUNK = 128
NCHK = E // CHUNK
BASE_CHK = NCHK // NW
EXTRA_W = NCHK - BASE_CHK * NW
N_PAD = 10240
ROWS_PER_TILE = N_PAD // NS
ZROWS = 64
CW = 16


def _seg_sum_sc(p, ei3, d, mul, with_counts, nbuf):
  mesh = plsc.VectorSubcoreMesh(
      core_axis_name="c", subcore_axis_name="s", num_cores=NC,
      num_subcores=NS)

  out_type = [jax.ShapeDtypeStruct((NC, N_PAD, 128), jnp.float32)]

  scratch = [
      pltpu.VMEM((BASE_CHK + 1, 2, CHUNK), jnp.int32),
      [pltpu.VMEM((CHUNK, d), jnp.float32) for _ in range(nbuf)],
      [pltpu.SemaphoreType.DMA for _ in range(nbuf)],
      pltpu.VMEM((ZROWS, d), jnp.float32),
      pltpu.VMEM_SHARED((N_PAD, d), jnp.float32),
      pltpu.SemaphoreType.DMA,
  ]
  if with_counts:
    scratch += [
        pltpu.VMEM((CHUNK, CW), jnp.float32),
        pltpu.VMEM((ZROWS, CW), jnp.float32),
        pltpu.VMEM_SHARED((N_PAD, CW), jnp.float32),
        pltpu.SemaphoreType.DMA,
    ]

  def body(p_hbm, ei_hbm, *rest):
    if with_counts:
      (out_hbm, ei_v, bufs, gsems, zer_v, acc_sh, isem,
       ones_v, zerc_v, cnt_sh, csem) = rest
    else:
      (out_hbm, ei_v, bufs, gsems, zer_v, acc_sh, isem) = rest
    sid = lax.axis_index("s")
    cid = lax.axis_index("c")
    wid = sid * NC + cid
    start_w = wid * BASE_CHK + jnp.minimum(wid, EXTRA_W)
    has_extra = wid < EXTRA_W
    my_chunks = jnp.where(has_extra, BASE_CHK + 1, BASE_CHK)

    pltpu.async_copy(ei_hbm.at[pl.ds(start_w, BASE_CHK)],
                     ei_v.at[pl.ds(0, BASE_CHK)], isem)

    @pl.when(has_extra)
    def _extra_row():
      pltpu.async_copy(ei_hbm.at[pl.ds(start_w + BASE_CHK, 1)],
                       ei_v.at[pl.ds(BASE_CHK, 1)], isem)

    def init_row(i, _):
      for j in range(d // 16):
        zer_v[i, pl.ds(j * 16, 16)] = jnp.zeros((16,), jnp.float32)
      if with_counts:
        zerc_v[i, pl.ds(0, 16)] = jnp.zeros((16,), jnp.float32)
      return _
    lax.fori_loop(0, ZROWS, init_row, 0)
    if with_counts:
      def init_ones(i, _):
        ones_v[i, pl.ds(0, 16)] = jnp.ones((16,), jnp.float32)
        return _
      lax.fori_loop(0, CHUNK, init_ones, 0)

    base = sid * ROWS_PER_TILE
    for k in range(ROWS_PER_TILE // ZROWS):
      pltpu.sync_copy(zer_v, acc_sh.at[pl.ds(base + k * ZROWS, ZROWS)])
      if with_counts:
        pltpu.sync_copy(zerc_v, cnt_sh.at[pl.ds(base + k * ZROWS, ZROWS)])

    pltpu.make_async_copy(ei_hbm.at[pl.ds(start_w, BASE_CHK)],
                          ei_v.at[pl.ds(0, BASE_CHK)], isem).wait()

    @pl.when(has_extra)
    def _extra_wait():
      pltpu.make_async_copy(ei_hbm.at[pl.ds(start_w + BASE_CHK, 1)],
                            ei_v.at[pl.ds(BASE_CHK, 1)], isem).wait()

    def scale_row(j, _):
      for t in range(CHUNK // 16):
        ei_v[j, 0, pl.ds(t * 16, 16)] = (
            ei_v[j, 0, pl.ds(t * 16, 16)] * mul)
      return _
    lax.fori_loop(0, my_chunks, scale_row, 0)

    plsc.subcore_barrier()

    def _start_gather(j, b):
      pltpu.async_copy(p_hbm.at[ei_v.at[j, 0]], bufs[b], gsems[b])

    for b in range(nbuf):
      _start_gather(b, b)

    def _drain(i, b):
      pltpu.make_async_copy(p_hbm.at[ei_v.at[0, 0]], bufs[b],
                            gsems[b]).wait()
      if with_counts:
        pltpu.async_copy(ones_v, cnt_sh.at[ei_v.at[i, 1]], csem)
      pltpu.sync_copy(bufs[b], acc_sh.at[ei_v.at[i, 1]], add=True)

    n_outer = BASE_CHK // nbuf
    n_tail = BASE_CHK - n_outer * nbuf

    def outer(g, _):
      for b in range(nbuf):
        i = g * nbuf + b
        _drain(i, b)

        @pl.when(i + nbuf < my_chunks)
        def _start():
          _start_gather(i + nbuf, b)
      return _
    lax.fori_loop(0, n_outer, outer, 0)
    for b in range(n_tail):
      _drain(n_outer * nbuf + b, (n_outer * nbuf + b) % nbuf)

    @pl.when(has_extra)
    def _extra_chunk():
      _drain(BASE_CHK, BASE_CHK % nbuf)

    if with_counts:
      def drain_cnt(i, _):
        pltpu.make_async_copy(ones_v, cnt_sh.at[ei_v.at[0, 1]],
                              csem).wait()
        return _
      lax.fori_loop(0, my_chunks, drain_cnt, 0)

    plsc.subcore_barrier()
    pltpu.sync_copy(acc_sh.at[pl.ds(base, ROWS_PER_TILE)],
                    out_hbm.at[cid, pl.ds(base, ROWS_PER_TILE),
                               pl.ds(0, d)])
    if with_counts:
      pltpu.sync_copy(cnt_sh.at[pl.ds(base, ROWS_PER_TILE)],
                      out_hbm.at[cid, pl.ds(base, ROWS_PER_TILE),
                                 pl.ds(64, CW)])

  fn = pl.kernel(
      body, out_type=out_type, mesh=mesh, scratch_types=scratch,
      compiler_params=pltpu.CompilerParams(use_tc_tiling_on_sc=False))
  return fn(p, ei3)


_BN = 2000


def _dot_t(a, w):
  return lax.dot_general(a, w, (((1,), (1,)), ((), ())),
                         preferred_element_type=jnp.float32)


def _tc_proj(x, w1l, w1r):
  def body(x_ref, wl_ref, wr_ref, pp_ref):
    xb = x_ref[...]
    pp_ref[...] = jnp.concatenate(
        [_dot_t(xb, wl_ref[...]), _dot_t(xb, wr_ref[...])], axis=1)

  return pl.pallas_call(
      body,
      grid=(N // _BN,),
      in_specs=[
          pl.BlockSpec((_BN, 128), lambda i: (i, 0)),
          pl.BlockSpec((64, 128), lambda i: (0, 0)),
          pl.BlockSpec((64, 128), lambda i: (0, 0)),
      ],
      out_specs=pl.BlockSpec((_BN, 128), lambda i: (i, 0)),
      out_shape=jax.ShapeDtypeStruct((N, 128), jnp.float32),
  )(x, w1l, w1r)


def _tc_mid(s1fat, pfat1, b1, w2l, w2r):
  def body(s_ref, p_ref, b_ref, wl_ref, wr_ref, o_ref):
    sf = s_ref[...]
    s = sf[0, :N, 0:64] + sf[1, :N, 0:64]
    cnt = sf[0, :N, 64:65] + sf[1, :N, 64:65]
    pf = p_ref[...]
    r1 = pf[:, 64:128]
    mean = s / jnp.maximum(cnt, 1.0)
    h1 = jnp.maximum(mean + b_ref[...] + r1, 0.0)
    p2 = _dot_t(h1, wl_ref[...])
    r2 = _dot_t(h1, wr_ref[...])
    cnt32 = jnp.broadcast_to(cnt, (N, 32))
    o_ref[...] = jnp.concatenate([p2, r2, cnt32, p2], axis=1)

  return pl.pallas_call(
      body,
      out_shape=jax.ShapeDtypeStruct((N, 128), jnp.float32),
  )(s1fat, pfat1, b1, w2l, w2r)


def _tc_final(s2fat, fat2, b2, wc, bc):
  def body(s_ref, f_ref, b_ref, wc_ref, bc_ref, h_ref, z_ref):
    sf = s_ref[...]
    s = sf[0, :N, 0:32] + sf[1, :N, 0:32]
    f2 = f_ref[...]
    r2 = f2[:, 32:64]
    cnt = f2[:, 64:65]
    mean = s / jnp.maximum(cnt, 1.0)
    h2t = jnp.transpose(jnp.maximum(mean + b_ref[...] + r2, 0.0))
    h_ref[...] = h2t
    z_ref[...] = (jnp.dot(wc_ref[...], h2t,
                          preferred_element_type=jnp.float32)
                  + bc_ref[...])

  return pl.pallas_call(
      body,
      out_shape=[
          jax.ShapeDtypeStruct((32, N), jnp.float32),
          jax.ShapeDtypeStruct((4, N), jnp.float32),
      ],
  )(s2fat, fat2, b2, wc, bc)


def kernel(x, edge_index, W1_l, b1_l, W1_r, W2_l, b2_l, W2_r, Wc, bc):
  ei3 = edge_index.reshape(2, NCHK, CHUNK).transpose(1, 0, 2)

  pfat1 = _tc_proj(x, W1_l, W1_r)
  s1fat = _seg_sum_sc(pfat1.reshape(2 * N, 64), ei3, 64, mul=2,
                      with_counts=True, nbuf=5)[0]
  fat2 = _tc_mid(s1fat, pfat1, b1_l.reshape(1, -1), W2_l, W2_r)
  s2fat = _seg_sum_sc(fat2.reshape(4 * N, 32), ei3, 32, mul=4,
                      with_counts=False, nbuf=12)[0]
  h2t, zt = _tc_final(s2fat, fat2, b2_l.reshape(1, -1), Wc,
                      bc.reshape(-1, 1))
  return (h2t.T, zt.T)
